# baseline (speedup 1.0000x reference)
import functools

import jax
import jax.numpy as jnp
from jax import lax
from jax.experimental import pallas as pl
from jax.experimental.pallas import tpu as pltpu
from jax.experimental.pallas import tpu_sc as plsc

BATCH = 16384
VALUE_RANGE = 8.0
A_RANGE = 3.0
DCONST = 1.702

_info = plsc.get_sparse_core_info()
_NC, _NS, _L = _info.num_cores, _info.num_subcores, _info.num_lanes
_NW = _NC * _NS
_CHUNK = BATCH // _NW


_NQ = 4
_QUARTER = _CHUNK // _NQ
_SCALE = DCONST * A_RANGE * VALUE_RANGE


def _body(user_h, item_h, th_h, a_h, b_h, c_h, out_h,
          uidx, iidx, thv, av, bv, cv, outv, semi, semg, semo):
    wid = lax.axis_index("s") * _NC + lax.axis_index("c")
    base = wid * _CHUNK
    cpu = pltpu.async_copy(user_h.at[pl.ds(base, _CHUNK)], uidx, semi)
    cpi = pltpu.async_copy(item_h.at[pl.ds(base, _CHUNK)], iidx, semi)
    cpu.wait()
    cpi.wait()

    def fire(lo, sem):
        s = pl.ds(lo, _QUARTER)
        return [
            pltpu.async_copy(th_h.at[0].at[uidx.at[s]], thv.at[s], sem),
            pltpu.async_copy(a_h.at[0].at[iidx.at[s]], av.at[s], sem),
            pltpu.async_copy(b_h.at[0].at[iidx.at[s]], bv.at[s], sem),
            pltpu.async_copy(c_h.at[0].at[iidx.at[s]], cv.at[s], sem),
        ]

    gs = [fire(q * _QUARTER, semg[q]) for q in range(_NQ)]

    def step(i, carry):
        s = pl.ds(i * _L, _L)
        et = jnp.exp(-thv[s])
        eb = jnp.exp(-bv[s])
        ea = jnp.exp(-av[s])
        ec = jnp.exp(-cv[s])
        z = _SCALE * (eb - et) / ((1.0 + ea) * ((1.0 + et) * (1.0 + eb)))
        w = jnp.exp(-z)
        outv[s] = (1.0 + w + ec) / ((1.0 + w) * (1.0 + ec))
        return carry

    nsteps = _QUARTER // _L
    ocs = []
    for q in range(_NQ):
        for cp in gs[q]:
            cp.wait()
        lax.fori_loop(q * nsteps, (q + 1) * nsteps, step, 0, unroll=4)
        ocs.append(pltpu.async_copy(
            outv.at[pl.ds(q * _QUARTER, _QUARTER)],
            out_h.at[pl.ds(base + q * _QUARTER, _QUARTER)], semo))
    for cp in ocs:
        cp.wait()


@jax.jit
def kernel(user, item, theta_w, a_w, b_w, c_w):
    run = pl.kernel(
        _body,
        out_type=jax.ShapeDtypeStruct((BATCH,), jnp.float32),
        mesh=plsc.VectorSubcoreMesh(core_axis_name="c", subcore_axis_name="s"),
        scratch_types=[
            pltpu.VMEM((_CHUNK,), jnp.int32),
            pltpu.VMEM((_CHUNK,), jnp.int32),
            pltpu.VMEM((_CHUNK,), jnp.float32),
            pltpu.VMEM((_CHUNK,), jnp.float32),
            pltpu.VMEM((_CHUNK,), jnp.float32),
            pltpu.VMEM((_CHUNK,), jnp.float32),
            pltpu.VMEM((_CHUNK,), jnp.float32),
            pltpu.SemaphoreType.DMA,
            [pltpu.SemaphoreType.DMA] * _NQ,
            pltpu.SemaphoreType.DMA,
        ],
    )
    return run(
        user.astype(jnp.int32),
        item.astype(jnp.int32),
        theta_w.reshape(1, -1),
        a_w.reshape(1, -1),
        b_w.reshape(1, -1),
        c_w.reshape(1, -1),
    )


# speedup vs baseline: 1.0364x; 1.0364x over previous
import functools

import jax
import jax.numpy as jnp
from jax import lax
from jax.experimental import pallas as pl
from jax.experimental.pallas import tpu as pltpu
from jax.experimental.pallas import tpu_sc as plsc

BATCH = 16384
VALUE_RANGE = 8.0
A_RANGE = 3.0
DCONST = 1.702

_info = plsc.get_sparse_core_info()
_NC, _NS, _L = _info.num_cores, _info.num_subcores, _info.num_lanes
_NW = _NC * _NS
_CHUNK = BATCH // _NW


_NQ = 2
_QUARTER = _CHUNK // _NQ
_SCALE = DCONST * A_RANGE * VALUE_RANGE


def _body(user_h, item_h, th_h, a_h, b_h, c_h, out_h,
          uidx, iidx, thv, av, bv, cv, outv, semi, semg, semo):
    wid = lax.axis_index("s") * _NC + lax.axis_index("c")
    base = wid * _CHUNK
    cpu = pltpu.async_copy(user_h.at[pl.ds(base, _CHUNK)], uidx, semi)
    cpi = pltpu.async_copy(item_h.at[pl.ds(base, _CHUNK)], iidx, semi)
    cpu.wait()
    cpi.wait()

    def fire(lo, sem):
        s = pl.ds(lo, _QUARTER)
        return [
            pltpu.async_copy(th_h.at[0].at[uidx.at[s]], thv.at[s], sem),
            pltpu.async_copy(a_h.at[0].at[iidx.at[s]], av.at[s], sem),
            pltpu.async_copy(b_h.at[0].at[iidx.at[s]], bv.at[s], sem),
            pltpu.async_copy(c_h.at[0].at[iidx.at[s]], cv.at[s], sem),
        ]

    gs = [fire(q * _QUARTER, semg[q]) for q in range(_NQ)]

    def step(i, carry):
        s = pl.ds(i * _L, _L)
        et = jnp.exp(-thv[s])
        eb = jnp.exp(-bv[s])
        ea = jnp.exp(-av[s])
        ec = jnp.exp(-cv[s])
        z = _SCALE * (eb - et) / ((1.0 + ea) * ((1.0 + et) * (1.0 + eb)))
        w = jnp.exp(-z)
        outv[s] = (1.0 + w + ec) / ((1.0 + w) * (1.0 + ec))
        return carry

    nsteps = _QUARTER // _L
    ocs = []
    for q in range(_NQ):
        for cp in gs[q]:
            cp.wait()
        lax.fori_loop(q * nsteps, (q + 1) * nsteps, step, 0, unroll=4)
        ocs.append(pltpu.async_copy(
            outv.at[pl.ds(q * _QUARTER, _QUARTER)],
            out_h.at[pl.ds(base + q * _QUARTER, _QUARTER)], semo))
    for cp in ocs:
        cp.wait()


@jax.jit
def kernel(user, item, theta_w, a_w, b_w, c_w):
    run = pl.kernel(
        _body,
        out_type=jax.ShapeDtypeStruct((BATCH,), jnp.float32),
        mesh=plsc.VectorSubcoreMesh(core_axis_name="c", subcore_axis_name="s"),
        scratch_types=[
            pltpu.VMEM((_CHUNK,), jnp.int32),
            pltpu.VMEM((_CHUNK,), jnp.int32),
            pltpu.VMEM((_CHUNK,), jnp.float32),
            pltpu.VMEM((_CHUNK,), jnp.float32),
            pltpu.VMEM((_CHUNK,), jnp.float32),
            pltpu.VMEM((_CHUNK,), jnp.float32),
            pltpu.VMEM((_CHUNK,), jnp.float32),
            pltpu.SemaphoreType.DMA,
            [pltpu.SemaphoreType.DMA] * _NQ,
            pltpu.SemaphoreType.DMA,
        ],
    )
    return run(
        user.astype(jnp.int32),
        item.astype(jnp.int32),
        theta_w.reshape(1, -1),
        a_w.reshape(1, -1),
        b_w.reshape(1, -1),
        c_w.reshape(1, -1),
    )


# speedup vs baseline: 1.0664x; 1.0289x over previous
import functools

import jax
import jax.numpy as jnp
from jax import lax
from jax.experimental import pallas as pl
from jax.experimental.pallas import tpu as pltpu
from jax.experimental.pallas import tpu_sc as plsc

BATCH = 16384
VALUE_RANGE = 8.0
A_RANGE = 3.0
DCONST = 1.702

_info = plsc.get_sparse_core_info()
_NC, _NS, _L = _info.num_cores, _info.num_subcores, _info.num_lanes
_NW = _NC * _NS
_CHUNK = BATCH // _NW


_NQ = 2
_QUARTER = _CHUNK // _NQ
_SCALE = DCONST * A_RANGE * VALUE_RANGE


def _body(user_h, item_h, th_h, a_h, b_h, c_h, out_h,
          uidx, iidx, thv, av, bv, cv, outv, semi, semg, semo):
    wid = lax.axis_index("s") * _NC + lax.axis_index("c")
    base = wid * _CHUNK
    cpu = pltpu.async_copy(user_h.at[pl.ds(base, _CHUNK)], uidx, semi)
    cpi = pltpu.async_copy(item_h.at[pl.ds(base, _CHUNK)], iidx, semi)

    def fire_theta(lo, sem):
        s = pl.ds(lo, _QUARTER)
        return pltpu.async_copy(th_h.at[0].at[uidx.at[s]], thv.at[s], sem)

    def fire_items(lo, sem):
        s = pl.ds(lo, _QUARTER)
        return [
            pltpu.async_copy(a_h.at[0].at[iidx.at[s]], av.at[s], sem),
            pltpu.async_copy(b_h.at[0].at[iidx.at[s]], bv.at[s], sem),
            pltpu.async_copy(c_h.at[0].at[iidx.at[s]], cv.at[s], sem),
        ]

    cpu.wait()
    gth = [fire_theta(q * _QUARTER, semg[q]) for q in range(_NQ)]
    cpi.wait()
    gs = [[gth[q]] + fire_items(q * _QUARTER, semg[q]) for q in range(_NQ)]

    def step(i, carry):
        s = pl.ds(i * _L, _L)
        et = jnp.exp(-thv[s])
        eb = jnp.exp(-bv[s])
        ea = jnp.exp(-av[s])
        ec = jnp.exp(-cv[s])
        z = _SCALE * (eb - et) / ((1.0 + ea) * ((1.0 + et) * (1.0 + eb)))
        w = jnp.exp(-z)
        outv[s] = (1.0 + w + ec) / ((1.0 + w) * (1.0 + ec))
        return carry

    nsteps = _QUARTER // _L
    ocs = []
    for q in range(_NQ):
        for cp in gs[q]:
            cp.wait()
        lax.fori_loop(q * nsteps, (q + 1) * nsteps, step, 0, unroll=2)
        ocs.append(pltpu.async_copy(
            outv.at[pl.ds(q * _QUARTER, _QUARTER)],
            out_h.at[pl.ds(base + q * _QUARTER, _QUARTER)], semo))
    for cp in ocs:
        cp.wait()


@jax.jit
def kernel(user, item, theta_w, a_w, b_w, c_w):
    run = pl.kernel(
        _body,
        out_type=jax.ShapeDtypeStruct((BATCH,), jnp.float32),
        mesh=plsc.VectorSubcoreMesh(core_axis_name="c", subcore_axis_name="s"),
        scratch_types=[
            pltpu.VMEM((_CHUNK,), jnp.int32),
            pltpu.VMEM((_CHUNK,), jnp.int32),
            pltpu.VMEM((_CHUNK,), jnp.float32),
            pltpu.VMEM((_CHUNK,), jnp.float32),
            pltpu.VMEM((_CHUNK,), jnp.float32),
            pltpu.VMEM((_CHUNK,), jnp.float32),
            pltpu.VMEM((_CHUNK,), jnp.float32),
            pltpu.SemaphoreType.DMA,
            [pltpu.SemaphoreType.DMA] * _NQ,
            pltpu.SemaphoreType.DMA,
        ],
    )
    return run(
        user.astype(jnp.int32),
        item.astype(jnp.int32),
        theta_w.reshape(1, -1),
        a_w.reshape(1, -1),
        b_w.reshape(1, -1),
        c_w.reshape(1, -1),
    )


# speedup vs baseline: 1.1153x; 1.0459x over previous
import functools

import jax
import jax.numpy as jnp
from jax import lax
from jax.experimental import pallas as pl
from jax.experimental.pallas import tpu as pltpu
from jax.experimental.pallas import tpu_sc as plsc

BATCH = 16384
VALUE_RANGE = 8.0
A_RANGE = 3.0
DCONST = 1.702

_info = plsc.get_sparse_core_info()
_NC, _NS, _L = _info.num_cores, _info.num_subcores, _info.num_lanes
_NW = _NC * _NS
_CHUNK = BATCH // _NW


_NQ = 2
_QUARTER = _CHUNK // _NQ
_SCALE = DCONST * A_RANGE * VALUE_RANGE


def _body(user_h, item_h, th_h, a_h, b_h, c_h, out_h,
          uidx, iidx, thv, av, bv, cv, outv, semi, semg, semo):
    wid = lax.axis_index("s") * _NC + lax.axis_index("c")
    base = wid * _CHUNK
    cpu = pltpu.async_copy(user_h.at[pl.ds(base, _CHUNK)], uidx, semi)
    cpi = pltpu.async_copy(item_h.at[pl.ds(base, _CHUNK)], iidx, semi)

    def fire_theta(lo, sem):
        s = pl.ds(lo, _QUARTER)
        return pltpu.async_copy(th_h.at[0].at[uidx.at[s]], thv.at[s], sem)

    def fire_items(lo, sem):
        s = pl.ds(lo, _QUARTER)
        return [
            pltpu.async_copy(a_h.at[0].at[iidx.at[s]], av.at[s], sem),
            pltpu.async_copy(b_h.at[0].at[iidx.at[s]], bv.at[s], sem),
            pltpu.async_copy(c_h.at[0].at[iidx.at[s]], cv.at[s], sem),
        ]

    cpu.wait()
    gth = [fire_theta(q * _QUARTER, semg[q]) for q in range(_NQ)]
    cpi.wait()
    gs = [[gth[q]] + fire_items(q * _QUARTER, semg[q]) for q in range(_NQ)]

    def step(i, carry):
        s = pl.ds(i * _L, _L)
        et = jnp.exp(-thv[s])
        eb = jnp.exp(-bv[s])
        ea = jnp.exp(-av[s])
        ec = jnp.exp(-cv[s])
        z = _SCALE * (eb - et) / ((1.0 + ea) * ((1.0 + et) * (1.0 + eb)))
        w = jnp.exp(-z)
        outv[s] = (1.0 + w + ec) / ((1.0 + w) * (1.0 + ec))
        return carry

    nsteps = _QUARTER // _L
    ocs = []
    for q in range(_NQ):
        for cp in gs[q]:
            cp.wait()
        lax.fori_loop(q * nsteps, (q + 1) * nsteps, step, 0, unroll=1)
        ocs.append(pltpu.async_copy(
            outv.at[pl.ds(q * _QUARTER, _QUARTER)],
            out_h.at[pl.ds(base + q * _QUARTER, _QUARTER)], semo))
    for cp in ocs:
        cp.wait()


@jax.jit
def kernel(user, item, theta_w, a_w, b_w, c_w):
    run = pl.kernel(
        _body,
        out_type=jax.ShapeDtypeStruct((BATCH,), jnp.float32),
        mesh=plsc.VectorSubcoreMesh(core_axis_name="c", subcore_axis_name="s"),
        scratch_types=[
            pltpu.VMEM((_CHUNK,), jnp.int32),
            pltpu.VMEM((_CHUNK,), jnp.int32),
            pltpu.VMEM((_CHUNK,), jnp.float32),
            pltpu.VMEM((_CHUNK,), jnp.float32),
            pltpu.VMEM((_CHUNK,), jnp.float32),
            pltpu.VMEM((_CHUNK,), jnp.float32),
            pltpu.VMEM((_CHUNK,), jnp.float32),
            pltpu.SemaphoreType.DMA,
            [pltpu.SemaphoreType.DMA] * _NQ,
            pltpu.SemaphoreType.DMA,
        ],
    )
    return run(
        user.astype(jnp.int32),
        item.astype(jnp.int32),
        theta_w.reshape(1, -1),
        a_w.reshape(1, -1),
        b_w.reshape(1, -1),
        c_w.reshape(1, -1),
    )
